# baseline (speedup 1.0000x reference)
import functools

import jax
import jax.numpy as jnp
from jax import lax
from jax.experimental import pallas as pl
from jax.experimental.pallas import tpu as pltpu
from jax.experimental.pallas import tpu_sc as plsc

E = 320000
D = 128
A = 16
NC = 2
NS = 16
NW = NC * NS
EPW = E // NW
CH = 40
NCHUNK = EPW // CH
NPAIR = NCHUNK // 2

_mesh = plsc.VectorSubcoreMesh(core_axis_name="c", subcore_axis_name="s")


@functools.partial(
    pl.kernel,
    out_type=jax.ShapeDtypeStruct((E, 2 * D + A), jnp.float32),
    mesh=_mesh,
    scratch_types=[
        pltpu.VMEM((NCHUNK, CH), jnp.int32),
        pltpu.VMEM((NCHUNK, CH), jnp.int32),
        pltpu.VMEM((CH, D), jnp.float32),
        pltpu.VMEM((CH, D), jnp.float32),
        pltpu.VMEM((CH, D), jnp.float32),
        pltpu.VMEM((CH, D), jnp.float32),
        pltpu.SemaphoreType.DMA,
        pltpu.SemaphoreType.DMA,
        pltpu.SemaphoreType.DMA,
        pltpu.SemaphoreType.DMA,
        pltpu.SemaphoreType.DMA,
    ],
)
def _edge_concat(x_hbm, ei_hbm, ea_hbm, out_hbm,
                 sidx, didx, sbuf0, sbuf1, dbuf0, dbuf1,
                 rsem0, rsem1, wsem0, wsem1, asem):
    wid = lax.axis_index("s") * NC + lax.axis_index("c")
    base = wid * EPW

    attr_cp = pltpu.async_copy(
        ea_hbm.at[pl.ds(base, EPW)],
        out_hbm.at[pl.ds(base, EPW), pl.ds(2 * D, A)], asem)

    pltpu.sync_copy(ei_hbm.at[0, wid], sidx)
    pltpu.sync_copy(ei_hbm.at[1, wid], didx)

    def issue_reads(j, sbuf, dbuf, rsem):
        pltpu.async_copy(x_hbm.at[sidx.at[j]], sbuf, rsem)
        pltpu.async_copy(x_hbm.at[didx.at[j]], dbuf, rsem)

    def wait_reads(sbuf, dbuf, rsem):
        pltpu.make_async_copy(x_hbm.at[sidx.at[0]], sbuf, rsem).wait()
        pltpu.make_async_copy(x_hbm.at[didx.at[0]], dbuf, rsem).wait()

    def issue_writes(j, sbuf, dbuf, wsem):
        gbase = base + j * CH
        pltpu.async_copy(sbuf, out_hbm.at[pl.ds(gbase, CH), pl.ds(0, D)], wsem)
        pltpu.async_copy(dbuf, out_hbm.at[pl.ds(gbase, CH), pl.ds(D, D)], wsem)

    def wait_writes(sbuf, dbuf, wsem):
        pltpu.make_async_copy(sbuf, out_hbm.at[pl.ds(base, CH), pl.ds(0, D)], wsem).wait()
        pltpu.make_async_copy(dbuf, out_hbm.at[pl.ds(base, CH), pl.ds(D, D)], wsem).wait()

    issue_reads(0, sbuf0, dbuf0, rsem0)

    def pair(g, carry):
        j0 = 2 * g
        j1 = j0 + 1

        @pl.when(g > 0)
        def _():
            wait_writes(sbuf1, dbuf1, wsem1)
        issue_reads(j1, sbuf1, dbuf1, rsem1)
        wait_reads(sbuf0, dbuf0, rsem0)
        issue_writes(j0, sbuf0, dbuf0, wsem0)

        @pl.when(g < NPAIR - 1)
        def _():
            wait_writes(sbuf0, dbuf0, wsem0)
            issue_reads(j0 + 2, sbuf0, dbuf0, rsem0)
        wait_reads(sbuf1, dbuf1, rsem1)
        issue_writes(j1, sbuf1, dbuf1, wsem1)
        return carry

    lax.fori_loop(0, NPAIR, pair, 0)

    wait_writes(sbuf0, dbuf0, wsem0)
    wait_writes(sbuf1, dbuf1, wsem1)
    attr_cp.wait()


def kernel(x, edge_index, edge_attr):
    ei = edge_index.astype(jnp.int32).reshape(2, NW, NCHUNK, CH)
    return _edge_concat(x, ei, edge_attr)


# speedup vs baseline: 6.5147x; 6.5147x over previous
import functools

import jax
import jax.numpy as jnp
from jax import lax
from jax.experimental import pallas as pl
from jax.experimental.pallas import tpu as pltpu
from jax.experimental.pallas import tpu_sc as plsc

E = 320000
D = 128
A = 16
NC = 2
NS = 16
NW = NC * NS
EPW = E // NW
CH = 40
NCHUNK = EPW // CH
NPAIR = NCHUNK // 2

_mesh = plsc.VectorSubcoreMesh(core_axis_name="c", subcore_axis_name="s")


@functools.partial(
    pl.kernel,
    out_type=jax.ShapeDtypeStruct((E, 2 * D + A), jnp.float32),
    mesh=_mesh,
    scratch_types=[
        pltpu.VMEM((NCHUNK, CH), jnp.int32),
        pltpu.VMEM((NCHUNK, CH), jnp.int32),
        pltpu.VMEM((CH, D), jnp.float32),
        pltpu.VMEM((CH, D), jnp.float32),
        pltpu.VMEM((CH, D), jnp.float32),
        pltpu.VMEM((CH, D), jnp.float32),
        pltpu.VMEM((CH, A), jnp.float32),
        pltpu.VMEM((CH, A), jnp.float32),
        pltpu.SemaphoreType.DMA,
        pltpu.SemaphoreType.DMA,
        pltpu.SemaphoreType.DMA,
        pltpu.SemaphoreType.DMA,
    ],
)
def _edge_concat(x_hbm, ei_hbm, ea_hbm, out_hbm,
                 sidx, didx, sbuf0, sbuf1, dbuf0, dbuf1, abuf0, abuf1,
                 rsem0, rsem1, wsem0, wsem1):
    wid = lax.axis_index("s") * NC + lax.axis_index("c")
    base = wid * EPW

    pltpu.sync_copy(ei_hbm.at[0, wid], sidx)
    pltpu.sync_copy(ei_hbm.at[1, wid], didx)

    def issue_reads(j, sbuf, dbuf, abuf, rsem):
        gbase = base + j * CH
        pltpu.async_copy(x_hbm.at[sidx.at[j]], sbuf, rsem)
        pltpu.async_copy(x_hbm.at[didx.at[j]], dbuf, rsem)
        pltpu.async_copy(ea_hbm.at[pl.ds(gbase, CH)], abuf, rsem)

    def wait_reads(sbuf, dbuf, abuf, rsem):
        pltpu.make_async_copy(x_hbm.at[sidx.at[0]], sbuf, rsem).wait()
        pltpu.make_async_copy(x_hbm.at[didx.at[0]], dbuf, rsem).wait()
        pltpu.make_async_copy(ea_hbm.at[pl.ds(base, CH)], abuf, rsem).wait()

    def issue_writes(j, sbuf, dbuf, abuf, wsem):
        gbase = base + j * CH
        pltpu.async_copy(sbuf, out_hbm.at[pl.ds(gbase, CH), pl.ds(0, D)], wsem)
        pltpu.async_copy(dbuf, out_hbm.at[pl.ds(gbase, CH), pl.ds(D, D)], wsem)
        pltpu.async_copy(abuf, out_hbm.at[pl.ds(gbase, CH), pl.ds(2 * D, A)], wsem)

    def wait_writes(sbuf, dbuf, abuf, wsem):
        pltpu.make_async_copy(sbuf, out_hbm.at[pl.ds(base, CH), pl.ds(0, D)], wsem).wait()
        pltpu.make_async_copy(dbuf, out_hbm.at[pl.ds(base, CH), pl.ds(D, D)], wsem).wait()
        pltpu.make_async_copy(abuf, out_hbm.at[pl.ds(base, CH), pl.ds(2 * D, A)], wsem).wait()

    issue_reads(0, sbuf0, dbuf0, abuf0, rsem0)

    def pair(g, carry):
        j0 = 2 * g
        j1 = j0 + 1

        @pl.when(g > 0)
        def _():
            wait_writes(sbuf1, dbuf1, abuf1, wsem1)
        issue_reads(j1, sbuf1, dbuf1, abuf1, rsem1)
        wait_reads(sbuf0, dbuf0, abuf0, rsem0)
        issue_writes(j0, sbuf0, dbuf0, abuf0, wsem0)

        @pl.when(g < NPAIR - 1)
        def _():
            wait_writes(sbuf0, dbuf0, abuf0, wsem0)
            issue_reads(j0 + 2, sbuf0, dbuf0, abuf0, rsem0)
        wait_reads(sbuf1, dbuf1, abuf1, rsem1)
        issue_writes(j1, sbuf1, dbuf1, abuf1, wsem1)
        return carry

    lax.fori_loop(0, NPAIR, pair, 0)

    wait_writes(sbuf0, dbuf0, abuf0, wsem0)
    wait_writes(sbuf1, dbuf1, abuf1, wsem1)


def kernel(x, edge_index, edge_attr):
    ei = edge_index.astype(jnp.int32).reshape(2, NW, NCHUNK, CH)
    return _edge_concat(x, ei, edge_attr)
